# baseline (speedup 1.0000x reference)
import functools

import jax
import jax.numpy as jnp
from jax import lax
from jax.experimental import pallas as pl
from jax.experimental.pallas import tpu as pltpu, tpu_sc as plsc

N_ITERS = 3
LAMBDA_INIT = 0.01

_info = plsc.get_sparse_core_info()
_NC, _NS = _info.num_cores, _info.num_subcores
_NW = _NC * _NS


def _make_gather(total_rows: int, table_len: int):
    rows_per_w = total_rows // _NW
    mesh = plsc.VectorSubcoreMesh(core_axis_name="c", subcore_axis_name="s")

    @functools.partial(
        pl.kernel,
        mesh=mesh,
        out_type=jax.ShapeDtypeStruct((total_rows, 128), jnp.float32),
        scratch_types=[
            pltpu.VMEM((rows_per_w, 128), jnp.int32),
            pltpu.VMEM((rows_per_w, 128), jnp.float32),
            pltpu.SemaphoreType.DMA,
        ],
    )
    def gk(table_hbm, idx_hbm, out_hbm, idx_v, rows_v, sem):
        wid = lax.axis_index("s") * _NC + lax.axis_index("c")
        base = wid * rows_per_w
        pltpu.sync_copy(idx_hbm.at[pl.ds(base, rows_per_w)], idx_v)

        def fire(j, carry):
            pltpu.async_copy(table_hbm.at[idx_v.at[j]], rows_v.at[j], sem)
            return carry

        lax.fori_loop(0, rows_per_w, fire, 0)
        pltpu.make_async_copy(out_hbm.at[pl.ds(base, rows_per_w)], rows_v, sem).wait()
        pltpu.sync_copy(rows_v, out_hbm.at[pl.ds(base, rows_per_w)])

    return gk


def _from_h(p):
    return p[..., :-1] / p[..., -1:]


def _skew(v):
    x, y, z = v[..., 0], v[..., 1], v[..., 2]
    o = jnp.zeros_like(x)
    M = jnp.stack([o, -z, y, z, o, -x, -y, x, o], axis=-1)
    return M.reshape(v.shape[:-1] + (3, 3))


def _so3exp(w):
    theta2 = jnp.sum(w * w)
    theta = jnp.sqrt(theta2 + 1e-12)
    W = _skew(w)
    A = jnp.sin(theta) / theta
    B = (1.0 - jnp.cos(theta)) / (theta2 + 1e-12)
    return jnp.eye(3, dtype=w.dtype) + A * W + B * (W @ W)


def _opt_step(g, H, lambda_, lr):
    D = jnp.diag(jnp.diag(H) + 1e-9)
    Hd = H + D * lambda_
    P = jnp.linalg.inv(Hd)
    return -lr * (P @ g[..., None])[..., 0]


def _project(R, t, pts3D, K):
    p_3d = (R @ pts3D.T).T + t
    proj = jnp.round(_from_h((K @ p_3d.T).T)).astype(jnp.int32) - 1
    return p_3d, proj


def kernel(pts3D, feature_ref, feature_map_query, feature_grad_x, feature_grad_y, K):
    N, C = feature_ref.shape
    Cm, H, W = feature_map_query.shape
    HW = H * W
    N_pad = ((N + 8 * _NW - 1) // (8 * _NW)) * (8 * _NW)

    fm_flat = feature_map_query.reshape(Cm * HW)
    gx_flat = feature_grad_x.reshape(Cm * HW)
    gy_flat = feature_grad_y.reshape(Cm * HW)

    total = Cm * N_pad
    total_rows = total // 128
    gather = _make_gather(total_rows, Cm * HW)
    chan_off = (jnp.arange(Cm, dtype=jnp.int32) * HW)[:, None]

    def gathered_feats(table_flat, rows, cols):
        r = jnp.clip(rows, 0, H - 1)
        c = jnp.clip(cols, 0, W - 1)
        hw = r * W + c
        hw_pad = jnp.pad(hw, (0, N_pad - N))
        idx = (chan_off + hw_pad[None, :]).reshape(total_rows, 128)
        out = gather(table_flat, idx).reshape(Cm, N_pad)
        return out[:, :N].T

    dtype = pts3D.dtype
    R = jnp.eye(3, dtype=dtype)
    t = jnp.array([1.0, 1.0, 0.0], dtype=dtype)
    lambda_ = jnp.asarray(LAMBDA_INIT, dtype=dtype)
    lr = jnp.asarray(0.1, dtype=dtype)
    lr_reset = 0.1
    prev_cost = None
    for i in range(N_ITERS):
        p_3d_1, proj = _project(R, t, pts3D, K)
        rows, cols = proj[:, 1], proj[:, 0]
        error = gathered_feats(fm_flat, rows, cols) - feature_ref
        cost = 0.5 * (error ** 2).sum(-1)
        if i == 0:
            prev_cost = cost.mean(-1)
        J_p_T = jnp.concatenate(
            [jnp.broadcast_to(jnp.eye(3, dtype=dtype), p_3d_1.shape[:-1] + (3, 3)),
             -_skew(p_3d_1)], axis=-1)
        shape = p_3d_1.shape[:-1]
        o = jnp.ones(shape, dtype=dtype)
        z = jnp.zeros(shape, dtype=dtype)
        J_px_p = jnp.stack(
            [K[0, 0] * o, z, -K[0, 0] * p_3d_1[..., 0] / p_3d_1[..., 2],
             z, K[1, 1] * o, -K[1, 1] * p_3d_1[..., 1] / p_3d_1[..., 2]],
            axis=-1).reshape(shape + (2, 3)) / p_3d_1[..., 2, None, None]
        grad_x_points = gathered_feats(gx_flat, rows, cols)
        grad_y_points = gathered_feats(gy_flat, rows, cols)
        J_f_px = jnp.concatenate(
            [grad_x_points[..., None], grad_y_points[..., None]], axis=-1)
        J_e_T = J_f_px @ J_px_p @ J_p_T
        Grad = jnp.einsum('bij,bi->bj', J_e_T, error).sum(-2)
        Hess = jnp.einsum('ijk,ijl->ikl', J_e_T, J_e_T).sum(-3)
        delta = _opt_step(Grad, Hess, lambda_, lr)
        dt, dw = delta[..., :3], delta[..., 3:6]
        dr = _so3exp(dw)
        R_new = dr @ R
        t_new = dr @ t + dt
        _, new_proj = _project(R_new, t_new, pts3D, K)
        new_error = gathered_feats(fm_flat, new_proj[:, 1], new_proj[:, 0]) - feature_ref
        new_cost = (0.5 * (new_error ** 2).sum(-1)).mean(-1)
        increased = new_cost > prev_cost
        lambda_ = jnp.clip(lambda_ * jnp.where(increased, 10.0, 0.1), 1e-6, 1e4)
        lr = jnp.where(increased, jnp.clip(0.1 * lr, 1e-3, 1.0), lr_reset)
        R = jnp.where(increased, R, R_new)
        t = jnp.where(increased, t, t_new)
        prev_cost = jnp.where(increased, prev_cost, new_cost)
    return R, t
